# baseline (speedup 1.0000x reference)
import jax
import jax.numpy as jnp
from jax import lax
from jax.experimental import pallas as pl
from jax.experimental.pallas import tpu as pltpu
from jax.experimental.pallas import tpu_sc as plsc

N_NODES = 10000
D = 128
E = 320000

_NC = 2
_NS = 16
_NW = _NC * _NS
_K = 80
_EPW = E // _NW
_CPW = _EPW // _K
_PLD = 1000
_NPAD = 10240
_RPS = _NPAD // _NS


def _sc_agg_body(x_hbm, src_hbm, dst_hbm, zrows_hbm,
                 sum_hbm, deg_hbm,
                 idx_s0, idx_d0, idx_c0, rows0, idx_s1, idx_d1, idx_c1, rows1,
                 sem0, sem1, semi0, semi1, sems0, sems1, deg_t, agg_s):
  cid = lax.axis_index("c")
  sid = lax.axis_index("s")
  wid = cid * _NS + sid

  r0 = pl.multiple_of(sid * _RPS, 8)
  pltpu.sync_copy(zrows_hbm, rows0)
  for j in range(_RPS // _K):
    pltpu.sync_copy(rows0, agg_s.at[pl.ds(r0 + j * _K, _K)])

  zeros16 = jnp.zeros((16,), jnp.float32)

  def zero_deg(i, carry):
    deg_t[pl.ds(i * 16, 16)] = zeros16
    return carry

  lax.fori_loop(0, _NPAD // 16, zero_deg, 0)

  zeros16i = jnp.zeros((16,), jnp.int32)
  for j in range(_K // 16):
    idx_c0[pl.ds(j * 16, 16)] = zeros16i
    idx_c1[pl.ds(j * 16, 16)] = zeros16i
  pltpu.sync_copy(zrows_hbm, rows1)
  plsc.subcore_barrier()

  e0 = wid * _EPW
  ones16 = jnp.ones((16,), jnp.float32)

  def start_idx(c, idx_s, idx_d, semi):
    base = pl.multiple_of(e0 + c * _K, 8)
    pltpu.async_copy(src_hbm.at[pl.ds(base, _K)], idx_s, semi)
    pltpu.async_copy(dst_hbm.at[pl.ds(base, _K)], idx_d, semi)

  def wait_idx(c, idx_s, idx_d, semi):
    base = pl.multiple_of(e0 + c * _K, 8)
    pltpu.make_async_copy(src_hbm.at[pl.ds(base, _K)], idx_s, semi).wait()
    pltpu.make_async_copy(dst_hbm.at[pl.ds(base, _K)], idx_d, semi).wait()

  def start_gather(idx_s, rows, sem):
    pltpu.async_copy(x_hbm.at[idx_s], rows, sem)

  def wait_gather(idx_s, rows, sem):
    pltpu.make_async_copy(x_hbm.at[idx_s], rows, sem).wait()

  def start_scatter(idx_c, rows, sems):
    pltpu.async_copy(rows, agg_s.at[idx_c], sems, add=True)

  def wait_scatter(idx_c, rows, sems):
    pltpu.make_async_copy(rows, agg_s.at[idx_c], sems).wait()

  def stage_and_degs(idx_d, idx_c):
    for j in range(_K // 16):
      dv = idx_d[pl.ds(j * 16, 16)]
      idx_c[pl.ds(j * 16, 16)] = dv
      plsc.addupdate_scatter(deg_t, [dv], ones16)

  start_scatter(idx_c0, rows0, sems0)
  start_scatter(idx_c1, rows1, sems1)
  start_idx(0, idx_s0, idx_d0, semi0)
  start_idx(1, idx_s1, idx_d1, semi1)

  def pair(g2, carry):
    c = g2 * 2
    wait_scatter(idx_c0, rows0, sems0)
    wait_idx(c, idx_s0, idx_d0, semi0)
    start_gather(idx_s0, rows0, sem0)
    wait_scatter(idx_c1, rows1, sems1)
    wait_idx(c + 1, idx_s1, idx_d1, semi1)
    start_gather(idx_s1, rows1, sem1)
    wait_gather(idx_s0, rows0, sem0)
    stage_and_degs(idx_d0, idx_c0)
    start_scatter(idx_c0, rows0, sems0)
    start_idx(c + 2, idx_s0, idx_d0, semi0)
    wait_gather(idx_s1, rows1, sem1)
    stage_and_degs(idx_d1, idx_c1)
    start_scatter(idx_c1, rows1, sems1)
    start_idx(c + 3, idx_s1, idx_d1, semi1)
    return carry

  lax.fori_loop(0, _CPW // 2, pair, 0)
  wait_scatter(idx_c0, rows0, sems0)
  wait_idx(_CPW - 1, idx_s0, idx_d0, semi0)
  start_gather(idx_s0, rows0, sem0)
  wait_gather(idx_s0, rows0, sem0)
  stage_and_degs(idx_d0, idx_c0)
  start_scatter(idx_c0, rows0, sems0)
  wait_idx(_CPW, idx_s1, idx_d1, semi1)
  wait_scatter(idx_c0, rows0, sems0)
  wait_scatter(idx_c1, rows1, sems1)
  plsc.subcore_barrier()

  o0 = pl.multiple_of(cid * _NPAD + sid * _RPS, 8)
  for j in range(_RPS // _K):
    pltpu.sync_copy(agg_s.at[pl.ds(r0 + j * _K, _K)], rows0)
    pltpu.sync_copy(rows0, sum_hbm.at[pl.ds(o0 + j * _K, _K)])
  t0 = pl.multiple_of(wid * _NPAD, 8)
  pltpu.sync_copy(deg_t, deg_hbm.at[pl.ds(t0, _NPAD)])


_sc_agg = pl.kernel(
    _sc_agg_body,
    out_type=[
        jax.ShapeDtypeStruct((_NC * _NPAD, D), jnp.float32),
        jax.ShapeDtypeStruct((_NW * _NPAD,), jnp.float32),
    ],
    mesh=plsc.VectorSubcoreMesh(core_axis_name="c", subcore_axis_name="s"),
    compiler_params=pltpu.CompilerParams(needs_layout_passes=False),
    scratch_types=[
        pltpu.VMEM((_K,), jnp.int32),
        pltpu.VMEM((_K,), jnp.int32),
        pltpu.VMEM((_K,), jnp.int32),
        pltpu.VMEM((_K, D), jnp.float32),
        pltpu.VMEM((_K,), jnp.int32),
        pltpu.VMEM((_K,), jnp.int32),
        pltpu.VMEM((_K,), jnp.int32),
        pltpu.VMEM((_K, D), jnp.float32),
        pltpu.SemaphoreType.DMA,
        pltpu.SemaphoreType.DMA,
        pltpu.SemaphoreType.DMA,
        pltpu.SemaphoreType.DMA,
        pltpu.SemaphoreType.DMA,
        pltpu.SemaphoreType.DMA,
        pltpu.VMEM((_NPAD,), jnp.float32),
        pltpu.VMEM_SHARED((_NPAD, D), jnp.float32),
    ],
)


def _dotT(a, w):
  return lax.dot_general(a, w, (((1,), (1,)), ((), ())),
                         preferred_element_type=jnp.float32)


def _mean_from_partials(s_ref, d_ref):
  deg = jnp.sum(d_ref[...], axis=1, keepdims=True)[:N_NODES]
  deg = jnp.maximum(deg, 1.0)
  return (s_ref[:N_NODES] + s_ref[_NPAD:_NPAD + N_NODES]) / deg


def _tc_layer1_body(x_ref, s_ref, d_ref, wl_ref, wr_ref, b_ref, g_ref, be_ref,
                    h_ref):
  mean = _mean_from_partials(s_ref, d_ref)
  t = _dotT(mean, wl_ref[...]) + _dotT(x_ref[...], wr_ref[...]) + b_ref[...]
  mu = jnp.mean(t, axis=0, keepdims=True)
  var = jnp.mean((t - mu) * (t - mu), axis=0, keepdims=True)
  h = g_ref[...] * (t - mu) * lax.rsqrt(var + 1e-5) + be_ref[...]
  h_ref[...] = jnp.maximum(h, 0.0)


def _tc_layer2_body(h_ref, s_ref, d_ref, wl_ref, wr_ref, b_ref, o_ref):
  mean = _mean_from_partials(s_ref, d_ref)
  o_ref[...] = _dotT(mean, wl_ref[...]) + _dotT(h_ref[...], wr_ref[...]) \
      + b_ref[...]


_tc_layer1 = pl.pallas_call(
    _tc_layer1_body,
    out_shape=jax.ShapeDtypeStruct((N_NODES, D), jnp.float32),
)

_tc_layer2 = pl.pallas_call(
    _tc_layer2_body,
    out_shape=jax.ShapeDtypeStruct((N_NODES, D), jnp.float32),
)


@jax.jit
def kernel(x, edge_index, W_l1, W_r1, b1, gamma1, beta1, W_l2, W_r2, b2):
  pad = jnp.zeros((2 * _K,), jnp.int32)
  src = jnp.concatenate([edge_index[0].astype(jnp.int32), pad])
  dst = jnp.concatenate([edge_index[1].astype(jnp.int32), pad])
  zrows = jnp.zeros((_K, D), jnp.float32)

  sum1, degp = _sc_agg(x, src, dst, zrows)
  degT = degp.reshape(_NW, _NPAD).T

  h = _tc_layer1(x, sum1, degT, W_l1, W_r1, b1.reshape(1, D),
                 gamma1.reshape(1, D), beta1.reshape(1, D))
  sum2, _ = _sc_agg(h, src, dst, zrows)
  out = _tc_layer2(h, sum2, degT, W_l2, W_r2, b2.reshape(1, D))
  return out


# speedup vs baseline: 1.1012x; 1.1012x over previous
import jax
import jax.numpy as jnp
from jax import lax
from jax.experimental import pallas as pl
from jax.experimental.pallas import tpu as pltpu
from jax.experimental.pallas import tpu_sc as plsc

N_NODES = 10000
D = 128
E = 320000

_NC = 2
_NS = 16
_NW = _NC * _NS
_K = 80
_EPW = E // _NW
_CPW = _EPW // _K
_PLD = 1000
_NPAD = 10240
_RPS = _NPAD // _NS


def _make_sc_body(do_deg):
  def body(x_hbm, src_hbm, dst_hbm, zrows_hbm, sum_hbm, *rest):
    if do_deg:
      (deg_hbm, idx_s0, idx_d0, idx_s1, idx_d1, idx_s2, idx_d2, idx_s3,
       idx_d3, rows0, rows1, sem0, sem1, semi0, semi1, semi2, semi3,
       deg_t, agg_s) = rest
    else:
      (idx_s0, idx_d0, idx_s1, idx_d1, idx_s2, idx_d2, idx_s3,
       idx_d3, rows0, rows1, sem0, sem1, semi0, semi1, semi2, semi3,
       agg_s) = rest
    cid = lax.axis_index("c")
    sid = lax.axis_index("s")
    wid = cid * _NS + sid

    r0 = pl.multiple_of(sid * _RPS, 8)
    pltpu.sync_copy(zrows_hbm, rows0)
    for j in range(_RPS // _K):
      pltpu.sync_copy(rows0, agg_s.at[pl.ds(r0 + j * _K, _K)])

    if do_deg:
      zeros16 = jnp.zeros((16,), jnp.float32)

      def zero_deg(i, carry):
        deg_t[pl.ds(i * 16, 16)] = zeros16
        return carry

      lax.fori_loop(0, _NPAD // 16, zero_deg, 0)
    plsc.subcore_barrier()

    e0 = wid * _EPW
    ones16 = jnp.ones((16,), jnp.float32)
    ib = ((idx_s0, idx_d0, semi0), (idx_s1, idx_d1, semi1),
          (idx_s2, idx_d2, semi2), (idx_s3, idx_d3, semi3))

    def start_idx(c, b):
      idx_s, idx_d, semi = ib[b]
      base = pl.multiple_of(e0 + c * _K, 8)
      pltpu.async_copy(src_hbm.at[pl.ds(base, _K)], idx_s, semi)
      pltpu.async_copy(dst_hbm.at[pl.ds(base, _K)], idx_d, semi)

    def wait_idx(c, b):
      idx_s, idx_d, semi = ib[b]
      base = pl.multiple_of(e0 + c * _K, 8)
      pltpu.make_async_copy(src_hbm.at[pl.ds(base, _K)], idx_s, semi).wait()
      pltpu.make_async_copy(dst_hbm.at[pl.ds(base, _K)], idx_d, semi).wait()

    def start_gather(b, rows, sem):
      pltpu.async_copy(x_hbm.at[ib[b][0]], rows, sem)

    def wait_gather(b, rows, sem):
      pltpu.make_async_copy(x_hbm.at[ib[b][0]], rows, sem).wait()

    def process(b, rows):
      idx_d = ib[b][1]
      pltpu.sync_copy(rows, agg_s.at[idx_d], add=True)
      if do_deg:
        for j in range(_K // 16):
          dv = idx_d[pl.ds(j * 16, 16)]
          plsc.addupdate_scatter(deg_t, [dv], ones16)

    for b in range(4):
      start_idx(b, b)

    def quad(q, carry):
      c = q * 4
      wait_idx(c, 0)
      start_gather(0, rows0, sem0)
      wait_idx(c + 1, 1)
      start_gather(1, rows1, sem1)
      wait_gather(0, rows0, sem0)
      process(0, rows0)
      start_idx(c + 4, 0)
      wait_idx(c + 2, 2)
      start_gather(2, rows0, sem0)
      wait_gather(1, rows1, sem1)
      process(1, rows1)
      start_idx(c + 5, 1)
      wait_idx(c + 3, 3)
      start_gather(3, rows1, sem1)
      wait_gather(2, rows0, sem0)
      process(2, rows0)
      start_idx(c + 6, 2)
      wait_gather(3, rows1, sem1)
      process(3, rows1)
      start_idx(c + 7, 3)
      return carry

    lax.fori_loop(0, _CPW // 4, quad, 0)
    wait_idx(_CPW - 1, 0)
    start_gather(0, rows0, sem0)
    wait_gather(0, rows0, sem0)
    process(0, rows0)
    wait_idx(_CPW, 1)
    wait_idx(_CPW + 1, 2)
    wait_idx(_CPW + 2, 3)
    plsc.subcore_barrier()

    o0 = pl.multiple_of(cid * _NPAD + sid * _RPS, 8)
    for j in range(_RPS // _K):
      pltpu.sync_copy(agg_s.at[pl.ds(r0 + j * _K, _K)], rows0)
      pltpu.sync_copy(rows0, sum_hbm.at[pl.ds(o0 + j * _K, _K)])
    if do_deg:
      t0 = pl.multiple_of(wid * _NPAD, 8)
      pltpu.sync_copy(deg_t, deg_hbm.at[pl.ds(t0, _NPAD)])

  return body


def _make_sc_kernel(do_deg):
  if do_deg:
    out_type = [jax.ShapeDtypeStruct((_NC * _NPAD, D), jnp.float32),
                jax.ShapeDtypeStruct((_NW * _NPAD,), jnp.float32)]
  else:
    out_type = jax.ShapeDtypeStruct((_NC * _NPAD, D), jnp.float32)
  scratch = ([pltpu.VMEM((_K,), jnp.int32)] * 8
             + [pltpu.VMEM((_K, D), jnp.float32)] * 2
             + [pltpu.SemaphoreType.DMA] * 6)
  if do_deg:
    scratch.append(pltpu.VMEM((_NPAD,), jnp.float32))
  scratch.append(pltpu.VMEM_SHARED((_NPAD, D), jnp.float32))
  return pl.kernel(
      _make_sc_body(do_deg),
      out_type=out_type,
      mesh=plsc.VectorSubcoreMesh(core_axis_name="c", subcore_axis_name="s"),
      compiler_params=pltpu.CompilerParams(needs_layout_passes=False),
      scratch_types=scratch,
  )


_sc_agg = _make_sc_kernel(True)
_sc_agg_nodeg = _make_sc_kernel(False)


def _dotT(a, w):
  return lax.dot_general(a, w, (((1,), (1,)), ((), ())),
                         preferred_element_type=jnp.float32)


def _mean_from_partials(s_ref, d_ref):
  deg = jnp.sum(d_ref[...], axis=1, keepdims=True)[:N_NODES]
  deg = jnp.maximum(deg, 1.0)
  return (s_ref[:N_NODES] + s_ref[_NPAD:_NPAD + N_NODES]) / deg


def _tc_layer1_body(x_ref, s_ref, d_ref, wl_ref, wr_ref, b_ref, g_ref, be_ref,
                    h_ref):
  mean = _mean_from_partials(s_ref, d_ref)
  t = _dotT(mean, wl_ref[...]) + _dotT(x_ref[...], wr_ref[...]) + b_ref[...]
  mu = jnp.mean(t, axis=0, keepdims=True)
  var = jnp.mean((t - mu) * (t - mu), axis=0, keepdims=True)
  h = g_ref[...] * (t - mu) * lax.rsqrt(var + 1e-5) + be_ref[...]
  h_ref[...] = jnp.maximum(h, 0.0)


def _tc_layer2_body(h_ref, s_ref, d_ref, wl_ref, wr_ref, b_ref, o_ref):
  mean = _mean_from_partials(s_ref, d_ref)
  o_ref[...] = _dotT(mean, wl_ref[...]) + _dotT(h_ref[...], wr_ref[...]) \
      + b_ref[...]


_tc_layer1 = pl.pallas_call(
    _tc_layer1_body,
    out_shape=jax.ShapeDtypeStruct((N_NODES, D), jnp.float32),
)

_tc_layer2 = pl.pallas_call(
    _tc_layer2_body,
    out_shape=jax.ShapeDtypeStruct((N_NODES, D), jnp.float32),
)


@jax.jit
def kernel(x, edge_index, W_l1, W_r1, b1, gamma1, beta1, W_l2, W_r2, b2):
  pad = jnp.zeros((4 * _K,), jnp.int32)
  src = jnp.concatenate([edge_index[0].astype(jnp.int32), pad])
  dst = jnp.concatenate([edge_index[1].astype(jnp.int32), pad])
  zrows = jnp.zeros((_K, D), jnp.float32)

  sum1, degp = _sc_agg(x, src, dst, zrows)
  degT = degp.reshape(_NW, _NPAD).T

  h = _tc_layer1(x, sum1, degT, W_l1, W_r1, b1.reshape(1, D),
                 gamma1.reshape(1, D), beta1.reshape(1, D))
  sum2 = _sc_agg_nodeg(h, src, dst, zrows)
  out = _tc_layer2(h, sum2, degT, W_l2, W_r2, b2.reshape(1, D))
  return out


# speedup vs baseline: 1.1053x; 1.0038x over previous
import jax
import jax.numpy as jnp
from jax import lax
from jax.experimental import pallas as pl
from jax.experimental.pallas import tpu as pltpu
from jax.experimental.pallas import tpu_sc as plsc

N_NODES = 10000
D = 128
E = 320000

_NC = 2
_NS = 16
_NW = _NC * _NS
_K = 80
_EPW = E // _NW
_CPW = _EPW // _K
_PLD = 1000
_NPAD = 10240
_RPS = _NPAD // _NS


def _make_sc_body(do_deg):
  def body(x_hbm, src_hbm, dst_hbm, zrows_hbm, sum_hbm, *rest):
    if do_deg:
      (deg_hbm, idx_s0, idx_d0, idx_s1, idx_d1, idx_s2, idx_d2, idx_s3,
       idx_d3, rows0, rows1, sem0, sem1, semi0, semi1, semi2, semi3,
       deg_t, agg_s) = rest
    else:
      (idx_s0, idx_d0, idx_s1, idx_d1, idx_s2, idx_d2, idx_s3,
       idx_d3, rows0, rows1, sem0, sem1, semi0, semi1, semi2, semi3,
       agg_s) = rest
    cid = lax.axis_index("c")
    sid = lax.axis_index("s")
    wid = cid * _NS + sid

    r0 = pl.multiple_of(sid * _RPS, 8)
    pltpu.sync_copy(zrows_hbm, rows0)
    for j in range(_RPS // _K):
      pltpu.sync_copy(rows0, agg_s.at[pl.ds(r0 + j * _K, _K)])

    if do_deg:
      zeros16 = jnp.zeros((16,), jnp.float32)

      def zero_deg(i, carry):
        deg_t[pl.ds(i * 16, 16)] = zeros16
        return carry

      lax.fori_loop(0, _NPAD // 16, zero_deg, 0)
    plsc.subcore_barrier()

    e0 = wid * _EPW
    ones16 = jnp.ones((16,), jnp.float32)
    ib = ((idx_s0, idx_d0, semi0), (idx_s1, idx_d1, semi1),
          (idx_s2, idx_d2, semi2), (idx_s3, idx_d3, semi3))

    def _idx_base(c):
      cc = jnp.minimum(c, _CPW - 1)
      return pl.multiple_of(e0 + cc * _K, 8)

    def start_idx(c, b):
      idx_s, idx_d, semi = ib[b]
      base = _idx_base(c)
      pltpu.async_copy(src_hbm.at[pl.ds(base, _K)], idx_s, semi)
      pltpu.async_copy(dst_hbm.at[pl.ds(base, _K)], idx_d, semi)

    def wait_idx(c, b):
      idx_s, idx_d, semi = ib[b]
      base = _idx_base(c)
      pltpu.make_async_copy(src_hbm.at[pl.ds(base, _K)], idx_s, semi).wait()
      pltpu.make_async_copy(dst_hbm.at[pl.ds(base, _K)], idx_d, semi).wait()

    def start_gather(b, rows, sem):
      pltpu.async_copy(x_hbm.at[ib[b][0]], rows, sem)

    def wait_gather(b, rows, sem):
      pltpu.make_async_copy(x_hbm.at[ib[b][0]], rows, sem).wait()

    def process(b, rows):
      idx_d = ib[b][1]
      pltpu.sync_copy(rows, agg_s.at[idx_d], add=True)
      if do_deg:
        for j in range(_K // 16):
          dv = idx_d[pl.ds(j * 16, 16)]
          plsc.addupdate_scatter(deg_t, [dv], ones16)

    for b in range(4):
      start_idx(b, b)

    def quad(q, carry):
      c = q * 4
      wait_idx(c, 0)
      start_gather(0, rows0, sem0)
      wait_idx(c + 1, 1)
      start_gather(1, rows1, sem1)
      wait_gather(0, rows0, sem0)
      process(0, rows0)
      start_idx(c + 4, 0)
      wait_idx(c + 2, 2)
      start_gather(2, rows0, sem0)
      wait_gather(1, rows1, sem1)
      process(1, rows1)
      start_idx(c + 5, 1)
      wait_idx(c + 3, 3)
      start_gather(3, rows1, sem1)
      wait_gather(2, rows0, sem0)
      process(2, rows0)
      start_idx(c + 6, 2)
      wait_gather(3, rows1, sem1)
      process(3, rows1)
      start_idx(c + 7, 3)
      return carry

    lax.fori_loop(0, _CPW // 4, quad, 0)
    wait_idx(_CPW - 1, 0)
    start_gather(0, rows0, sem0)
    wait_gather(0, rows0, sem0)
    process(0, rows0)
    wait_idx(_CPW, 1)
    wait_idx(_CPW + 1, 2)
    wait_idx(_CPW + 2, 3)
    plsc.subcore_barrier()

    o0 = pl.multiple_of(cid * _NPAD + sid * _RPS, 8)
    for j in range(_RPS // _K):
      pltpu.sync_copy(agg_s.at[pl.ds(r0 + j * _K, _K)], rows0)
      pltpu.sync_copy(rows0, sum_hbm.at[pl.ds(o0 + j * _K, _K)])
    if do_deg:
      t0 = pl.multiple_of(wid * _NPAD, 8)
      pltpu.sync_copy(deg_t, deg_hbm.at[pl.ds(t0, _NPAD)])

  return body


def _make_sc_kernel(do_deg):
  if do_deg:
    out_type = [jax.ShapeDtypeStruct((_NC * _NPAD, D), jnp.float32),
                jax.ShapeDtypeStruct((_NW * _NPAD,), jnp.float32)]
  else:
    out_type = jax.ShapeDtypeStruct((_NC * _NPAD, D), jnp.float32)
  scratch = ([pltpu.VMEM((_K,), jnp.int32)] * 8
             + [pltpu.VMEM((_K, D), jnp.float32)] * 2
             + [pltpu.SemaphoreType.DMA] * 6)
  if do_deg:
    scratch.append(pltpu.VMEM((_NPAD,), jnp.float32))
  scratch.append(pltpu.VMEM_SHARED((_NPAD, D), jnp.float32))
  return pl.kernel(
      _make_sc_body(do_deg),
      out_type=out_type,
      mesh=plsc.VectorSubcoreMesh(core_axis_name="c", subcore_axis_name="s"),
      compiler_params=pltpu.CompilerParams(needs_layout_passes=False),
      scratch_types=scratch,
  )


_sc_agg = _make_sc_kernel(True)
_sc_agg_nodeg = _make_sc_kernel(False)


def _dotT(a, w):
  return lax.dot_general(a, w, (((1,), (1,)), ((), ())),
                         preferred_element_type=jnp.float32)


def _mean_from_partials(s_ref, d_ref):
  deg = lax.dot_general(d_ref[...], jnp.ones((_NW, 1), jnp.float32),
                        (((0,), (0,)), ((), ())),
                        preferred_element_type=jnp.float32)[:N_NODES]
  deg = jnp.maximum(deg, 1.0)
  return (s_ref[:N_NODES] + s_ref[_NPAD:_NPAD + N_NODES]) / deg


def _tc_layer1_body(x_ref, s_ref, d_ref, wl_ref, wr_ref, b_ref, g_ref, be_ref,
                    h_ref):
  mean = _mean_from_partials(s_ref, d_ref)
  t = _dotT(mean, wl_ref[...]) + _dotT(x_ref[...], wr_ref[...]) + b_ref[...]
  mu = jnp.mean(t, axis=0, keepdims=True)
  var = jnp.mean((t - mu) * (t - mu), axis=0, keepdims=True)
  h = g_ref[...] * (t - mu) * lax.rsqrt(var + 1e-5) + be_ref[...]
  h_ref[...] = jnp.maximum(h, 0.0)


def _tc_layer2_body(h_ref, s_ref, d_ref, wl_ref, wr_ref, b_ref, o_ref):
  mean = _mean_from_partials(s_ref, d_ref)
  o_ref[...] = _dotT(mean, wl_ref[...]) + _dotT(h_ref[...], wr_ref[...]) \
      + b_ref[...]


_tc_layer1 = pl.pallas_call(
    _tc_layer1_body,
    out_shape=jax.ShapeDtypeStruct((N_NODES, D), jnp.float32),
)

_tc_layer2 = pl.pallas_call(
    _tc_layer2_body,
    out_shape=jax.ShapeDtypeStruct((N_NODES, D), jnp.float32),
)


@jax.jit
def kernel(x, edge_index, W_l1, W_r1, b1, gamma1, beta1, W_l2, W_r2, b2):
  src = edge_index[0].astype(jnp.int32)
  dst = edge_index[1].astype(jnp.int32)
  zrows = jnp.zeros((_K, D), jnp.float32)

  sum1, degp = _sc_agg(x, src, dst, zrows)
  degT = degp.reshape(_NW, _NPAD)

  h = _tc_layer1(x, sum1, degT, W_l1, W_r1, b1.reshape(1, D),
                 gamma1.reshape(1, D), beta1.reshape(1, D))
  sum2 = _sc_agg_nodeg(h, src, dst, zrows)
  out = _tc_layer2(h, sum2, degT, W_l2, W_r2, b2.reshape(1, D))
  return out


# speedup vs baseline: 1.1354x; 1.0272x over previous
import jax
import jax.numpy as jnp
from jax import lax
from jax.experimental import pallas as pl
from jax.experimental.pallas import tpu as pltpu
from jax.experimental.pallas import tpu_sc as plsc

N_NODES = 10000
D = 128
E = 320000

_NC = 2
_NS = 16
_NW = _NC * _NS
_K = 80
_EPW = E // _NW
_CPW = _EPW // _K
_PLD = 1000
_NPAD = 10240
_RPS = _NPAD // _NS


def _make_sc_body(do_deg):
  def body(x_hbm, src_hbm, dst_hbm, zrows_hbm, sum_hbm, *rest):
    if do_deg:
      (deg_hbm, idx_s0, idx_d0, idx_s1, idx_d1, idx_s2, idx_d2, idx_s3,
       idx_d3, rows0, rows1, sem0, sem1, semi0, semi1, semi2, semi3,
       deg_t, agg_s) = rest
    else:
      (idx_s0, idx_d0, idx_s1, idx_d1, idx_s2, idx_d2, idx_s3,
       idx_d3, rows0, rows1, sem0, sem1, semi0, semi1, semi2, semi3,
       agg_s) = rest
    cid = lax.axis_index("c")
    sid = lax.axis_index("s")
    wid = cid * _NS + sid

    r0 = pl.multiple_of(sid * _RPS, 8)
    pltpu.sync_copy(zrows_hbm, rows0)
    for j in range(_RPS // _K):
      pltpu.async_copy(rows0, agg_s.at[pl.ds(r0 + j * _K, _K)], sem0)

    if do_deg:
      zeros16 = jnp.zeros((16,), jnp.float32)

      def zero_deg(i, carry):
        deg_t[pl.ds(i * 16, 16)] = zeros16
        return carry

      lax.fori_loop(0, _NPAD // 16, zero_deg, 0)
    for j in range(_RPS // _K):
      pltpu.make_async_copy(rows0, agg_s.at[pl.ds(r0 + j * _K, _K)],
                            sem0).wait()
    plsc.subcore_barrier()

    e0 = wid * _EPW
    ones16 = jnp.ones((16,), jnp.float32)
    ib = ((idx_s0, idx_d0, semi0), (idx_s1, idx_d1, semi1),
          (idx_s2, idx_d2, semi2), (idx_s3, idx_d3, semi3))

    def _idx_base(c):
      cc = jnp.minimum(c, _CPW - 1)
      return pl.multiple_of(e0 + cc * _K, 8)

    def start_idx(c, b):
      idx_s, idx_d, semi = ib[b]
      base = _idx_base(c)
      pltpu.async_copy(src_hbm.at[pl.ds(base, _K)], idx_s, semi)
      pltpu.async_copy(dst_hbm.at[pl.ds(base, _K)], idx_d, semi)

    def wait_idx(c, b):
      idx_s, idx_d, semi = ib[b]
      base = _idx_base(c)
      pltpu.make_async_copy(src_hbm.at[pl.ds(base, _K)], idx_s, semi).wait()
      pltpu.make_async_copy(dst_hbm.at[pl.ds(base, _K)], idx_d, semi).wait()

    def start_gather(b, rows, sem):
      pltpu.async_copy(x_hbm.at[ib[b][0]], rows, sem)

    def wait_gather(b, rows, sem):
      pltpu.make_async_copy(x_hbm.at[ib[b][0]], rows, sem).wait()

    def process(b, rows):
      idx_d = ib[b][1]
      pltpu.sync_copy(rows, agg_s.at[idx_d], add=True)
      if do_deg:
        for j in range(_K // 16):
          dv = idx_d[pl.ds(j * 16, 16)]
          plsc.addupdate_scatter(deg_t, [dv], ones16)

    for b in range(4):
      start_idx(b, b)

    def quad(q, carry):
      c = q * 4
      wait_idx(c, 0)
      start_gather(0, rows0, sem0)
      wait_idx(c + 1, 1)
      start_gather(1, rows1, sem1)
      wait_gather(0, rows0, sem0)
      process(0, rows0)
      start_idx(c + 4, 0)
      wait_idx(c + 2, 2)
      start_gather(2, rows0, sem0)
      wait_gather(1, rows1, sem1)
      process(1, rows1)
      start_idx(c + 5, 1)
      wait_idx(c + 3, 3)
      start_gather(3, rows1, sem1)
      wait_gather(2, rows0, sem0)
      process(2, rows0)
      start_idx(c + 6, 2)
      wait_gather(3, rows1, sem1)
      process(3, rows1)
      start_idx(c + 7, 3)
      return carry

    lax.fori_loop(0, _CPW // 4, quad, 0)
    wait_idx(_CPW - 1, 0)
    start_gather(0, rows0, sem0)
    wait_gather(0, rows0, sem0)
    process(0, rows0)
    wait_idx(_CPW, 1)
    wait_idx(_CPW + 1, 2)
    wait_idx(_CPW + 2, 3)
    plsc.subcore_barrier()

    o0 = pl.multiple_of(cid * _NPAD + sid * _RPS, 8)
    if do_deg:
      t0 = pl.multiple_of(wid * _NPAD, 8)
      pltpu.async_copy(deg_t, deg_hbm.at[pl.ds(t0, _NPAD)], semi0)
    db = (rows0, rows1)
    ds_ = (sem0, sem1)
    for j in range(_RPS // _K):
      buf, semb = db[j % 2], ds_[j % 2]
      if j >= 2:
        jp = j - 2
        pltpu.make_async_copy(buf, sum_hbm.at[pl.ds(o0 + jp * _K, _K)],
                              semb).wait()
      pltpu.sync_copy(agg_s.at[pl.ds(r0 + j * _K, _K)], buf)
      pltpu.async_copy(buf, sum_hbm.at[pl.ds(o0 + j * _K, _K)], semb)
    for j in range(_RPS // _K - 2, _RPS // _K):
      buf, semb = db[j % 2], ds_[j % 2]
      pltpu.make_async_copy(buf, sum_hbm.at[pl.ds(o0 + j * _K, _K)],
                            semb).wait()
    if do_deg:
      pltpu.make_async_copy(deg_t, deg_hbm.at[pl.ds(t0, _NPAD)],
                            semi0).wait()

  return body


def _make_sc_kernel(do_deg):
  if do_deg:
    out_type = [jax.ShapeDtypeStruct((_NC * _NPAD, D), jnp.float32),
                jax.ShapeDtypeStruct((_NW * _NPAD,), jnp.float32)]
  else:
    out_type = jax.ShapeDtypeStruct((_NC * _NPAD, D), jnp.float32)
  scratch = ([pltpu.VMEM((_K,), jnp.int32)] * 8
             + [pltpu.VMEM((_K, D), jnp.float32)] * 2
             + [pltpu.SemaphoreType.DMA] * 6)
  if do_deg:
    scratch.append(pltpu.VMEM((_NPAD,), jnp.float32))
  scratch.append(pltpu.VMEM_SHARED((_NPAD, D), jnp.float32))
  return pl.kernel(
      _make_sc_body(do_deg),
      out_type=out_type,
      mesh=plsc.VectorSubcoreMesh(core_axis_name="c", subcore_axis_name="s"),
      compiler_params=pltpu.CompilerParams(needs_layout_passes=False),
      scratch_types=scratch,
  )


_sc_agg = _make_sc_kernel(True)
_sc_agg_nodeg = _make_sc_kernel(False)


def _dotT(a, w):
  return lax.dot_general(a, w, (((1,), (1,)), ((), ())),
                         preferred_element_type=jnp.float32)


def _mean_from_partials(s_ref, d_ref):
  deg = lax.dot_general(d_ref[...], jnp.ones((_NW, 1), jnp.float32),
                        (((0,), (0,)), ((), ())),
                        preferred_element_type=jnp.float32)[:N_NODES]
  deg = jnp.maximum(deg, 1.0)
  return (s_ref[:N_NODES] + s_ref[_NPAD:_NPAD + N_NODES]) / deg


def _tc_layer1_body(x_ref, s_ref, d_ref, wl_ref, wr_ref, b_ref, g_ref, be_ref,
                    h_ref):
  mean = _mean_from_partials(s_ref, d_ref)
  t = _dotT(mean, wl_ref[...]) + _dotT(x_ref[...], wr_ref[...]) + b_ref[...]
  mu = jnp.mean(t, axis=0, keepdims=True)
  var = jnp.mean((t - mu) * (t - mu), axis=0, keepdims=True)
  h = g_ref[...] * (t - mu) * lax.rsqrt(var + 1e-5) + be_ref[...]
  h_ref[...] = jnp.maximum(h, 0.0)


def _tc_layer2_body(h_ref, s_ref, d_ref, wl_ref, wr_ref, b_ref, o_ref):
  mean = _mean_from_partials(s_ref, d_ref)
  o_ref[...] = _dotT(mean, wl_ref[...]) + _dotT(h_ref[...], wr_ref[...]) \
      + b_ref[...]


_tc_layer1 = pl.pallas_call(
    _tc_layer1_body,
    out_shape=jax.ShapeDtypeStruct((N_NODES, D), jnp.float32),
)

_tc_layer2 = pl.pallas_call(
    _tc_layer2_body,
    out_shape=jax.ShapeDtypeStruct((N_NODES, D), jnp.float32),
)


@jax.jit
def kernel(x, edge_index, W_l1, W_r1, b1, gamma1, beta1, W_l2, W_r2, b2):
  src = edge_index[0].astype(jnp.int32)
  dst = edge_index[1].astype(jnp.int32)
  zrows = jnp.zeros((_K, D), jnp.float32)

  sum1, degp = _sc_agg(x, src, dst, zrows)
  degT = degp.reshape(_NW, _NPAD)

  h = _tc_layer1(x, sum1, degT, W_l1, W_r1, b1.reshape(1, D),
                 gamma1.reshape(1, D), beta1.reshape(1, D))
  sum2 = _sc_agg_nodeg(h, src, dst, zrows)
  out = _tc_layer2(h, sum2, degT, W_l2, W_r2, b2.reshape(1, D))
  return out
